# baseline (speedup 1.0000x reference)
import functools

import jax
import jax.numpy as jnp
from jax import lax
from jax.experimental import pallas as pl
from jax.experimental.pallas import tpu as pltpu
from jax.experimental.pallas import tpu_sc as plsc

N_NODES = 50000
LANES = 16
TBLK = 12800
N_PAD = 50048
QUARTER = N_PAD // 4
SUBQ = QUARTER // 2
CHUNK = 8000


def _sc_body(E, EP, NCHUNK,
             dst_hbm, lgT_hbm, scT_hbm,
             out_hbm, part_hbm, fin_hbm,
             table, dst_buf, lg_buf, sc_buf, out_buf):
    c = lax.axis_index("c")
    s = lax.axis_index("s")
    head_local = s // 4
    head = c * 4 + head_local
    part = s % 4
    w = c * 16 + s

    def init_body(i, _):
        table[pl.ds(i * LANES, LANES)] = jnp.full((LANES,), -jnp.inf, jnp.float32)
        return 0
    lax.fori_loop(0, N_PAD // LANES, init_body, 0)

    def chunk1(ci, _):
        base = pl.multiple_of(part * EP + ci * CHUNK, 8)
        pltpu.sync_copy(dst_hbm.at[pl.ds(base, CHUNK)], dst_buf)
        pltpu.sync_copy(lgT_hbm.at[pl.ds(head * E + base, CHUNK)], lg_buf)

        def vec(i, _):
            d = dst_buf[pl.ds(i * LANES, LANES)]
            v = lg_buf[pl.ds(i * LANES, LANES)]
            g = plsc.load_gather(table, [d])

            def cond(gc):
                return jnp.any(v > gc)

            def wbody(gc):
                plsc.store_scatter(table, [d], v, mask=v > gc)
                return plsc.load_gather(table, [d])

            lax.while_loop(cond, wbody, g)
            return 0
        lax.fori_loop(0, CHUNK // LANES, vec, 0)
        return 0
    lax.fori_loop(0, NCHUNK, chunk1, 0)

    pltpu.sync_copy(table, part_hbm.at[pl.ds(w * N_PAD, N_PAD)])
    plsc.subcore_barrier()

    team = c * 16 + head_local * 4
    for q2 in range(2):
        qoff = part * QUARTER + q2 * SUBQ
        pltpu.sync_copy(part_hbm.at[pl.ds(team * N_PAD + qoff, SUBQ)],
                        lg_buf.at[pl.ds(0, SUBQ)])
        for j in range(1, 4):
            pltpu.sync_copy(part_hbm.at[pl.ds((team + j) * N_PAD + qoff, SUBQ)],
                            sc_buf.at[pl.ds(0, SUBQ)])

            def mx_body(i, _):
                sl = pl.ds(i * LANES, LANES)
                lg_buf[sl] = jnp.maximum(lg_buf[sl], sc_buf[sl])
                return 0
            lax.fori_loop(0, SUBQ // LANES, mx_body, 0)
        pltpu.sync_copy(lg_buf.at[pl.ds(0, SUBQ)],
                        fin_hbm.at[pl.ds(head * N_PAD + qoff, SUBQ)])
    plsc.subcore_barrier()
    pltpu.sync_copy(fin_hbm.at[pl.ds(head * N_PAD, N_PAD)], table)

    def chunk3(ci, _):
        base = pl.multiple_of(part * EP + ci * CHUNK, 8)
        pltpu.sync_copy(dst_hbm.at[pl.ds(base, CHUNK)], dst_buf)
        pltpu.sync_copy(lgT_hbm.at[pl.ds(head * E + base, CHUNK)], lg_buf)
        pltpu.sync_copy(scT_hbm.at[pl.ds(head * E + base, CHUNK)], sc_buf)

        def vec(i, _):
            sl = pl.ds(i * LANES, LANES)
            d = dst_buf[sl]
            mx = plsc.load_gather(table, [d])
            out_buf[sl] = sc_buf[sl] * jnp.exp(lg_buf[sl] - mx)
            return 0
        lax.fori_loop(0, CHUNK // LANES, vec, 0)
        pltpu.sync_copy(out_buf, out_hbm.at[pl.ds(head * E + base, CHUNK)])
        return 0
    lax.fori_loop(0, NCHUNK, chunk3, 0)


def _fwd_tr_body(lg_ref, sc_ref, lgT_ref, scT_ref):
    eye = jnp.eye(8, dtype=jnp.float32)
    dn = (((1,), (1,)), ((), ()))
    lgT_ref[...] = lax.dot_general(eye, lg_ref[...], dn,
                                   precision=lax.Precision.HIGHEST,
                                   preferred_element_type=jnp.float32)
    scT_ref[...] = lax.dot_general(eye, sc_ref[...], dn,
                                   precision=lax.Precision.HIGHEST,
                                   preferred_element_type=jnp.float32)


def _bwd_tr_body(eT_ref, out_ref):
    eye = jnp.eye(8, dtype=jnp.float32)
    dn = (((0,), (0,)), ((), ()))
    out_ref[...] = lax.dot_general(eT_ref[...], eye, dn,
                                   precision=lax.Precision.HIGHEST,
                                   preferred_element_type=jnp.float32)


def _transpose_in(lg2d, sc2d, E):
    grid = E // TBLK
    return pl.pallas_call(
        _fwd_tr_body,
        grid=(grid,),
        in_specs=[pl.BlockSpec((TBLK, 8), lambda i: (i, 0)),
                  pl.BlockSpec((TBLK, 8), lambda i: (i, 0))],
        out_specs=[pl.BlockSpec((8, TBLK), lambda i: (0, i)),
                   pl.BlockSpec((8, TBLK), lambda i: (0, i))],
        out_shape=[jax.ShapeDtypeStruct((8, E), jnp.float32),
                   jax.ShapeDtypeStruct((8, E), jnp.float32)],
    )(lg2d, sc2d)


def _transpose_out(eT2d, E):
    grid = E // TBLK
    return pl.pallas_call(
        _bwd_tr_body,
        grid=(grid,),
        in_specs=[pl.BlockSpec((8, TBLK), lambda i: (0, i))],
        out_specs=pl.BlockSpec((TBLK, 8), lambda i: (i, 0)),
        out_shape=jax.ShapeDtypeStruct((E, 8), jnp.float32),
    )(eT2d)


def kernel(edge_index, logits, scale):
    E, H = scale.shape
    assert H == 8 and E % (4 * CHUNK) == 0 and E % TBLK == 0
    EP = E // 4
    NCHUNK = EP // CHUNK

    dst = edge_index[1]
    lgT2d, scT2d = _transpose_in(logits.reshape(E, H), scale, E)
    lgT = lgT2d.reshape(-1)
    scT = scT2d.reshape(-1)

    mesh = plsc.VectorSubcoreMesh(core_axis_name="c", subcore_axis_name="s")
    body = functools.partial(_sc_body, E, EP, NCHUNK)
    eT, _parts, _fin = pl.kernel(
        body,
        out_type=(
            jax.ShapeDtypeStruct((H * E,), jnp.float32),
            jax.ShapeDtypeStruct((32 * N_PAD,), jnp.float32),
            jax.ShapeDtypeStruct((8 * N_PAD,), jnp.float32),
        ),
        mesh=mesh,
        compiler_params=pltpu.CompilerParams(needs_layout_passes=False),
        scratch_types=[
            pltpu.VMEM((N_PAD,), jnp.float32),
            pltpu.VMEM((CHUNK,), jnp.int32),
            pltpu.VMEM((CHUNK,), jnp.float32),
            pltpu.VMEM((CHUNK,), jnp.float32),
            pltpu.VMEM((CHUNK,), jnp.float32),
        ],
    )(dst, lgT, scT)

    return _transpose_out(eT.reshape(H, E), E).reshape(E, H, 1)


# speedup vs baseline: 2.1622x; 2.1622x over previous
import functools

import jax
import jax.numpy as jnp
from jax import lax
from jax.experimental import pallas as pl
from jax.experimental.pallas import tpu as pltpu
from jax.experimental.pallas import tpu_sc as plsc

N_NODES = 50000
LANES = 16
N_PAD = 50048
QUARTER = N_PAD // 4
SUBQ = QUARTER // 2
CHUNK1 = 2000
CHUNK2 = 2000
FINSLICE = 8 * N_PAD // 16


def _seg_max_body(E, EP, NC1,
                  dst_hbm, lg_hbm, part_hbm, fin_hbm,
                  table, dst_buf, lgf_buf):
    c = lax.axis_index("c")
    s = lax.axis_index("s")
    head_local = s // 4
    head = c * 4 + head_local
    part = s % 4
    w = c * 16 + s

    def init_body(i, _):
        table[pl.ds(i * LANES, LANES)] = jnp.full((LANES,), -jnp.inf, jnp.float32)
        return 0
    lax.fori_loop(0, N_PAD // LANES, init_body, 0)

    pre = lax.iota(jnp.int32, LANES) * 8 + head

    def chunk1(ci, _):
        ebase = pl.multiple_of(part * EP + ci * CHUNK1, 8)
        pltpu.sync_copy(dst_hbm.at[pl.ds(ebase, CHUNK1)], dst_buf)
        pltpu.sync_copy(lg_hbm.at[pl.ds(ebase * 8, CHUNK1 * 8)], lgf_buf)

        def vec(j, _):
            d = dst_buf[pl.ds(j * LANES, LANES)]
            v = plsc.load_gather(lgf_buf, [pre + j * 128])
            g = plsc.load_gather(table, [d])

            def cond(gc):
                return jnp.any(v > gc)

            def wbody(gc):
                plsc.store_scatter(table, [d], v, mask=v > gc)
                return plsc.load_gather(table, [d])

            lax.while_loop(cond, wbody, g)
            return 0
        lax.fori_loop(0, CHUNK1 // LANES, vec, 0)
        return 0
    lax.fori_loop(0, NC1, chunk1, 0)

    pltpu.sync_copy(table, part_hbm.at[pl.ds(w * N_PAD, N_PAD)])
    plsc.subcore_barrier()

    team = c * 16 + head_local * 4
    for q2 in range(2):
        qoff = part * QUARTER + q2 * SUBQ
        pltpu.sync_copy(part_hbm.at[pl.ds(team * N_PAD + qoff, SUBQ)],
                        lgf_buf.at[pl.ds(0, SUBQ)])
        for j in range(1, 4):
            pltpu.sync_copy(part_hbm.at[pl.ds((team + j) * N_PAD + qoff, SUBQ)],
                            lgf_buf.at[pl.ds(8000, SUBQ)])

            def mx_body(i, _):
                a = pl.ds(i * LANES, LANES)
                b = pl.ds(8000 + i * LANES, LANES)
                lgf_buf[a] = jnp.maximum(lgf_buf[a], lgf_buf[b])
                return 0
            lax.fori_loop(0, SUBQ // LANES, mx_body, 0)
        pltpu.sync_copy(lgf_buf.at[pl.ds(0, SUBQ)],
                        fin_hbm.at[pl.ds(head * N_PAD + qoff, SUBQ)])


def _softmax_body(E, EP2, NC2,
                  dst_hbm, lg_hbm, sc_hbm, fin_hbm, out_hbm,
                  tabs, dst_buf, lgf_buf, scf_buf, outf_buf, idx_buf, mx_buf,
                  sem):
    c = lax.axis_index("c")
    s = lax.axis_index("s")
    w = c * 16 + s

    half = FINSLICE // 2
    for hh in range(2):
        fslice = pl.multiple_of(s * FINSLICE + hh * half, 8)
        pltpu.sync_copy(fin_hbm.at[pl.ds(fslice, half)],
                        mx_buf.at[pl.ds(0, half)])
        pltpu.sync_copy(mx_buf.at[pl.ds(0, half)],
                        tabs.at[pl.ds(fslice, half)])
    plsc.subcore_barrier()

    iot = lax.iota(jnp.int32, LANES)
    esel = iot // 8
    hsel = iot - esel * 8

    def chunk2(ci, _):
        ebase = pl.multiple_of(w * EP2 + ci * CHUNK2, 8)
        pltpu.sync_copy(dst_hbm.at[pl.ds(ebase, CHUNK2)], dst_buf)
        pltpu.sync_copy(lg_hbm.at[pl.ds(ebase * 8, CHUNK2 * 8)], lgf_buf)
        pltpu.sync_copy(sc_hbm.at[pl.ds(ebase * 8, CHUNK2 * 8)], scf_buf)

        def build_idx(j, _):
            d16 = plsc.load_gather(dst_buf, [esel + j * 2])
            idx_buf[pl.ds(j * LANES, LANES)] = d16 + hsel * N_PAD
            return 0
        lax.fori_loop(0, CHUNK2 * 8 // LANES, build_idx, 0)

        pltpu.async_copy(tabs.at[idx_buf], mx_buf, sem).wait()

        def vec(j, _):
            sl = pl.ds(j * LANES, LANES)
            outf_buf[sl] = scf_buf[sl] * jnp.exp(lgf_buf[sl] - mx_buf[sl])
            return 0
        lax.fori_loop(0, CHUNK2 * 8 // LANES, vec, 0)
        pltpu.sync_copy(outf_buf, out_hbm.at[pl.ds(ebase * 8, CHUNK2 * 8)])
        return 0
    lax.fori_loop(0, NC2, chunk2, 0)


def kernel(edge_index, logits, scale):
    E, H = scale.shape
    assert H == 8 and E % (4 * CHUNK1) == 0 and E % (32 * CHUNK2) == 0
    EP = E // 4
    NC1 = EP // CHUNK1
    EP2 = E // 32
    NC2 = EP2 // CHUNK2

    dst = edge_index[1]
    lgflat = logits.reshape(E * H)
    scflat = scale.reshape(E * H)

    mesh = plsc.VectorSubcoreMesh(core_axis_name="c", subcore_axis_name="s")
    params = pltpu.CompilerParams(needs_layout_passes=False)

    _parts, fin = pl.kernel(
        functools.partial(_seg_max_body, E, EP, NC1),
        out_type=(
            jax.ShapeDtypeStruct((32 * N_PAD,), jnp.float32),
            jax.ShapeDtypeStruct((8 * N_PAD,), jnp.float32),
        ),
        mesh=mesh,
        compiler_params=params,
        scratch_types=[
            pltpu.VMEM((N_PAD,), jnp.float32),
            pltpu.VMEM((CHUNK1,), jnp.int32),
            pltpu.VMEM((CHUNK1 * 8,), jnp.float32),
        ],
    )(dst, lgflat)

    eflat = pl.kernel(
        functools.partial(_softmax_body, E, EP2, NC2),
        out_type=jax.ShapeDtypeStruct((E * H,), jnp.float32),
        mesh=mesh,
        compiler_params=params,
        scratch_types=[
            pltpu.VMEM_SHARED((8 * N_PAD,), jnp.float32),
            pltpu.VMEM((CHUNK2,), jnp.int32),
            pltpu.VMEM((CHUNK2 * 8,), jnp.float32),
            pltpu.VMEM((CHUNK2 * 8,), jnp.float32),
            pltpu.VMEM((CHUNK2 * 8,), jnp.float32),
            pltpu.VMEM((CHUNK2 * 8,), jnp.int32),
            pltpu.VMEM((CHUNK2 * 8,), jnp.float32),
            pltpu.SemaphoreType.DMA,
        ],
    )(dst, lgflat, scflat, fin)

    return eflat.reshape(E, H, 1)


# speedup vs baseline: 3.9094x; 1.8081x over previous
import functools

import jax
import jax.numpy as jnp
from jax import lax
from jax.experimental import pallas as pl
from jax.experimental.pallas import tpu as pltpu
from jax.experimental.pallas import tpu_sc as plsc

N_NODES = 50000
LANES = 16
N_PAD = 50048
QUARTER = N_PAD // 4
SUBQ = QUARTER // 2
CHUNK = 16000
CBLK = CHUNK // 128


def _sc_body(E, EP, NCH,
             dst_hbm, lgT_hbm, scB_hbm,
             out_hbm, part_hbm, fin_hbm,
             table, dst_buf, lg_buf, sc_buf, out_buf, red_b):
    c = lax.axis_index("c")
    s = lax.axis_index("s")
    head_local = s // 4
    head = c * 4 + head_local
    part = s % 4
    w = c * 16 + s

    def init_body(i, _):
        table[pl.ds(i * LANES, LANES)] = jnp.full((LANES,), -jnp.inf, jnp.float32)
        return 0
    lax.fori_loop(0, N_PAD // LANES, init_body, 0)

    def chunk1(ci, _):
        base = pl.multiple_of(part * EP + ci * CHUNK, 128)
        pltpu.sync_copy(dst_hbm.at[pl.ds(base, CHUNK)], dst_buf)
        pltpu.sync_copy(lgT_hbm.at[pl.ds(head * E + base, CHUNK)], lg_buf)

        def vec(j, _):
            d = dst_buf[pl.ds(j * LANES, LANES)]
            v = lg_buf[pl.ds(j * LANES, LANES)]
            g = plsc.load_gather(table, [d])

            def cond(gc):
                return jnp.any(v > gc)

            def wbody(gc):
                plsc.store_scatter(table, [d], v, mask=v > gc)
                return plsc.load_gather(table, [d])

            lax.while_loop(cond, wbody, g)
            return 0
        lax.fori_loop(0, CHUNK // LANES, vec, 0)
        return 0
    lax.fori_loop(0, NCH, chunk1, 0)

    pltpu.sync_copy(table, part_hbm.at[pl.ds(w * N_PAD, N_PAD)])
    plsc.subcore_barrier()

    team = c * 16 + head_local * 4
    for q2 in range(2):
        qoff = part * QUARTER + q2 * SUBQ
        pltpu.sync_copy(part_hbm.at[pl.ds(team * N_PAD + qoff, SUBQ)],
                        out_buf.at[pl.ds(0, SUBQ)])
        for j in range(1, 4):
            pltpu.sync_copy(part_hbm.at[pl.ds((team + j) * N_PAD + qoff, SUBQ)],
                            red_b)

            def mx_body(i, _):
                sl = pl.ds(i * LANES, LANES)
                out_buf[sl] = jnp.maximum(out_buf[sl], red_b[sl])
                return 0
            lax.fori_loop(0, SUBQ // LANES, mx_body, 0)
        pltpu.sync_copy(out_buf.at[pl.ds(0, SUBQ)],
                        fin_hbm.at[pl.ds(head * N_PAD + qoff, SUBQ)])
    plsc.subcore_barrier()
    pltpu.sync_copy(fin_hbm.at[pl.ds(head * N_PAD, N_PAD)], table)

    def chunk3(ci, _):
        base = pl.multiple_of(part * EP + ci * CHUNK, 128)
        bblk = part * (EP // 128) + ci * CBLK
        pltpu.sync_copy(dst_hbm.at[pl.ds(base, CHUNK)], dst_buf)
        pltpu.sync_copy(lgT_hbm.at[pl.ds(head * E + base, CHUNK)], lg_buf)
        pltpu.sync_copy(scB_hbm.at[pl.ds(bblk, CBLK), head, :], sc_buf)

        def vec(j, _):
            sl = pl.ds(j * LANES, LANES)
            d = dst_buf[sl]
            mx = plsc.load_gather(table, [d])
            sc = sc_buf[j // 8, pl.ds((j % 8) * LANES, LANES)]
            out_buf[sl] = sc * jnp.exp(lg_buf[sl] - mx)
            return 0
        lax.fori_loop(0, CHUNK // LANES, vec, 0)
        pltpu.sync_copy(out_buf, out_hbm.at[pl.ds(head * E + base, CHUNK)])
        return 0
    lax.fori_loop(0, NCH, chunk3, 0)


def kernel(edge_index, logits, scale):
    E, H = scale.shape
    assert H == 8 and E % (4 * CHUNK) == 0 and E % 128 == 0
    EP = E // 4
    NCH = EP // CHUNK

    dst = edge_index[1]
    lgT = logits.transpose(1, 0, 2).reshape(H * E)
    scB = scale.reshape(E // 128, 128, H).transpose(0, 2, 1)

    mesh = plsc.VectorSubcoreMesh(core_axis_name="c", subcore_axis_name="s")
    params = pltpu.CompilerParams(needs_layout_passes=False)

    eT, _parts, _fin = pl.kernel(
        functools.partial(_sc_body, E, EP, NCH),
        out_type=(
            jax.ShapeDtypeStruct((H * E,), jnp.float32),
            jax.ShapeDtypeStruct((32 * N_PAD,), jnp.float32),
            jax.ShapeDtypeStruct((8 * N_PAD,), jnp.float32),
        ),
        mesh=mesh,
        compiler_params=params,
        scratch_types=[
            pltpu.VMEM((N_PAD,), jnp.float32),
            pltpu.VMEM((CHUNK,), jnp.int32),
            pltpu.VMEM((CHUNK,), jnp.float32),
            pltpu.VMEM((CBLK, 128), jnp.float32),
            pltpu.VMEM((CHUNK,), jnp.float32),
            pltpu.VMEM((SUBQ,), jnp.float32),
        ],
    )(dst, lgT, scB)

    return eT.reshape(1, H, E).transpose(2, 1, 0)


# speedup vs baseline: 7.2653x; 1.8584x over previous
import functools

import jax
import jax.numpy as jnp
from jax import lax
from jax.experimental import pallas as pl
from jax.experimental.pallas import tpu as pltpu
from jax.experimental.pallas import tpu_sc as plsc

N_NODES = 50000
LANES = 16
N_PAD = 50048
QUARTER = N_PAD // 4
SUBQ = QUARTER // 2
CHUNK = 3200
CBLK = CHUNK // 128
FUSE = 4


def _sc_body(E, EP, NCH,
             dst_hbm, lgT_hbm, scB_hbm,
             out_hbm, part_hbm, fin_hbm,
             table, table_b, dst_buf, lg_buf, sc_buf, out_buf, red_b):
    c = lax.axis_index("c")
    s = lax.axis_index("s")
    head_local = s // 4
    head = c * 4 + head_local
    part = s % 4
    w = c * 16 + s

    def init_body(i, _):
        ninf = jnp.full((LANES,), -jnp.inf, jnp.float32)
        table[pl.ds(i * LANES, LANES)] = ninf
        table_b[pl.ds(i * LANES, LANES)] = ninf
        return 0
    lax.fori_loop(0, N_PAD // LANES, init_body, 0)

    def chunk1(ci, _):
        base = pl.multiple_of(part * EP + ci * CHUNK, 128)
        pltpu.sync_copy(dst_hbm.at[pl.ds(base, CHUNK)], dst_buf)
        pltpu.sync_copy(lgT_hbm.at[pl.ds(head * E + base, CHUNK)], lg_buf)

        tabsel = [table, table_b, table, table_b]

        def vec(j4, _):
            j0 = j4 * FUSE
            ds_ = [dst_buf[pl.ds((j0 + k) * LANES, LANES)] for k in range(FUSE)]
            vs = [lg_buf[pl.ds((j0 + k) * LANES, LANES)] for k in range(FUSE)]
            gs = tuple(plsc.load_gather(tabsel[k], [ds_[k]]) for k in range(FUSE))

            def cond(gc):
                m = vs[0] > gc[0]
                for k in range(1, FUSE):
                    m = m | (vs[k] > gc[k])
                return jnp.any(m)

            def wbody(gc):
                for k in range(FUSE):
                    plsc.store_scatter(tabsel[k], [ds_[k]], vs[k],
                                       mask=vs[k] > gc[k])
                return tuple(plsc.load_gather(tabsel[k], [ds_[k]])
                             for k in range(FUSE))

            lax.while_loop(cond, wbody, gs)
            return 0
        lax.fori_loop(0, CHUNK // LANES // FUSE, vec, 0)
        return 0
    lax.fori_loop(0, NCH, chunk1, 0)

    pltpu.sync_copy(table, part_hbm.at[pl.ds(w * N_PAD, N_PAD)])
    pltpu.sync_copy(table_b, part_hbm.at[pl.ds((32 + w) * N_PAD, N_PAD)])
    plsc.subcore_barrier()

    team = c * 16 + head_local * 4
    rows = [team + j for j in range(4)] + [32 + team + j for j in range(4)]
    for q2 in range(2):
        qoff = part * QUARTER + q2 * SUBQ
        pltpu.sync_copy(part_hbm.at[pl.ds(rows[0] * N_PAD + qoff, SUBQ)],
                        out_buf.at[pl.ds(0, SUBQ)])
        for r in rows[1:]:
            pltpu.sync_copy(part_hbm.at[pl.ds(r * N_PAD + qoff, SUBQ)],
                            red_b)

            def mx_body(i, _):
                sl = pl.ds(i * LANES, LANES)
                out_buf[sl] = jnp.maximum(out_buf[sl], red_b[sl])
                return 0
            lax.fori_loop(0, SUBQ // LANES, mx_body, 0)
        pltpu.sync_copy(out_buf.at[pl.ds(0, SUBQ)],
                        fin_hbm.at[pl.ds(head * N_PAD + qoff, SUBQ)])
    plsc.subcore_barrier()
    pltpu.sync_copy(fin_hbm.at[pl.ds(head * N_PAD, N_PAD)], table)

    def chunk3(ci, _):
        base = pl.multiple_of(part * EP + ci * CHUNK, 128)
        bblk = part * (EP // 128) + ci * CBLK
        pltpu.sync_copy(dst_hbm.at[pl.ds(base, CHUNK)], dst_buf)
        pltpu.sync_copy(lgT_hbm.at[pl.ds(head * E + base, CHUNK)], lg_buf)
        pltpu.sync_copy(scB_hbm.at[pl.ds(bblk, CBLK), head, :], sc_buf)

        @plsc.parallel_loop(0, CHUNK // LANES, unroll=4)
        def vec(j):
            sl = pl.ds(j * LANES, LANES)
            d = dst_buf[sl]
            mx = plsc.load_gather(table, [d])
            sc = sc_buf[j // 8, pl.ds((j % 8) * LANES, LANES)]
            out_buf[sl] = sc * jnp.exp(lg_buf[sl] - mx)
        pltpu.sync_copy(out_buf, out_hbm.at[pl.ds(head * E + base, CHUNK)])
        return 0
    lax.fori_loop(0, NCH, chunk3, 0)


def kernel(edge_index, logits, scale):
    E, H = scale.shape
    assert H == 8 and E % (4 * CHUNK) == 0 and E % 128 == 0
    EP = E // 4
    NCH = EP // CHUNK

    dst = edge_index[1]
    lgT = logits.transpose(1, 0, 2).reshape(H * E)
    scB = scale.reshape(E // 128, 128, H).transpose(0, 2, 1)

    mesh = plsc.VectorSubcoreMesh(core_axis_name="c", subcore_axis_name="s")
    params = pltpu.CompilerParams(needs_layout_passes=False)

    eT, _parts, _fin = pl.kernel(
        functools.partial(_sc_body, E, EP, NCH),
        out_type=(
            jax.ShapeDtypeStruct((H * E,), jnp.float32),
            jax.ShapeDtypeStruct((64 * N_PAD,), jnp.float32),
            jax.ShapeDtypeStruct((8 * N_PAD,), jnp.float32),
        ),
        mesh=mesh,
        compiler_params=params,
        scratch_types=[
            pltpu.VMEM((N_PAD,), jnp.float32),
            pltpu.VMEM((N_PAD,), jnp.float32),
            pltpu.VMEM((CHUNK,), jnp.int32),
            pltpu.VMEM((CHUNK,), jnp.float32),
            pltpu.VMEM((CBLK, 128), jnp.float32),
            pltpu.VMEM((CHUNK,), jnp.float32),
            pltpu.VMEM((SUBQ,), jnp.float32),
        ],
    )(dst, lgT, scB)

    return eT.reshape(1, H, E).transpose(2, 1, 0)
